# Initial kernel scaffold; baseline (speedup 1.0000x reference)
import jax
import jax.numpy as jnp
from jax.experimental import pallas as pl


def kernel(context_vector, embeddings, thetas):
    raise NotImplementedError("write your pallas kernel here")


# speedup vs baseline: 1.2367x; 1.2367x over previous
import functools

import jax
import jax.numpy as jnp
from jax import lax
from jax.experimental import pallas as pl
from jax.experimental.pallas import tpu as pltpu
from jax.experimental.pallas import tpu_sc as plsc

VOCAB = 100000
D = 128
DEPTH = 17
N_INTERNAL = 2**DEPTH - 1
B = 4096
CTX = 8

_INFO = plsc.get_sparse_core_info()
NC, NS, L = _INFO.num_cores, _INFO.num_subcores, _INFO.num_lanes
NW = NC * NS
BW = B // NW
NCHUNK = BW // L
NG = BW // L

_mesh = plsc.VectorSubcoreMesh(core_axis_name="c", subcore_axis_name="s")


@functools.partial(
    pl.kernel,
    out_type=[
        jax.ShapeDtypeStruct((B,), jnp.int32),
        jax.ShapeDtypeStruct((NW, DEPTH, BW), jnp.float32),
    ],
    mesh=_mesh,
    compiler_params=pltpu.CompilerParams(needs_layout_passes=False),
    scratch_types=[
        pltpu.VMEM((L * CTX, D), jnp.float32),
        pltpu.VMEM((D, BW), jnp.float32),
        pltpu.VMEM((BW, D), jnp.float32),
        pltpu.VMEM((L * CTX,), jnp.int32),
        pltpu.VMEM((BW,), jnp.int32),
        pltpu.VMEM((DEPTH, BW), jnp.float32),
        pltpu.SemaphoreType.DMA,
    ],
)
def _hs_kernel(ctx_hbm, emb_hbm, th_hbm, leaf_hbm, scores_hbm,
               rows_v, xwT_v, theta_v, idx_v, nodes_v, scoresT_v, sem):
    wid = lax.axis_index("s") * NC + lax.axis_index("c")
    base = pl.multiple_of(wid * BW, BW)
    lanes = jnp.arange(L, dtype=jnp.int32)

    def chunk_body(c, carry):
        off = pl.multiple_of((base + c * L) * CTX, L * CTX)
        pltpu.sync_copy(ctx_hbm.at[pl.ds(off, L * CTX)], idx_v)
        pltpu.async_copy(emb_hbm.at[idx_v], rows_v, sem).wait()
        row0 = lanes * CTX

        def d_body(d, carry2):
            col = jnp.full((L,), d, jnp.int32)
            acc = plsc.load_gather(rows_v, [row0, col])
            for j in range(1, CTX):
                acc = acc + plsc.load_gather(rows_v, [row0 + j, col])
            xwT_v[d, pl.ds(c * L, L)] = acc
            return carry2

        return lax.fori_loop(0, D, d_body, carry)

    lax.fori_loop(0, NCHUNK, chunk_body, 0)

    for g in range(NG):
        nodes_v[pl.ds(g * L, L)] = jnp.zeros((L,), jnp.int32)

    def step_body(t, carry):
        pltpu.async_copy(th_hbm.at[nodes_v], theta_v, sem).wait()

        def group_body(g, carry2):
            gds = pl.ds(g * L, L)
            rowg = g * L + lanes
            zero = jnp.zeros((L,), jnp.float32)

            def d_body(dd, accs):
                d0 = dd * 8
                out = []
                for k in range(8):
                    col = jnp.full((L,), d0 + k, jnp.int32)
                    th = plsc.load_gather(theta_v, [rowg, col])
                    out.append(accs[k] + xwT_v[d0 + k, gds] * th)
                return tuple(out)

            accs = lax.fori_loop(0, D // 8, d_body, (zero,) * 8)
            score = ((accs[0] + accs[1]) + (accs[2] + accs[3])) + (
                (accs[4] + accs[5]) + (accs[6] + accs[7]))
            scoresT_v[t, gds] = score
            nd = nodes_v[gds]
            nodes_v[gds] = 2 * nd + jnp.where(score < 0.0, 1, 2)
            return carry2

        return lax.fori_loop(0, NG, group_body, carry)

    lax.fori_loop(0, DEPTH, step_body, 0)

    for g in range(NG):
        gds = pl.ds(g * L, L)
        nodes_v[gds] = nodes_v[gds] - N_INTERNAL
    pltpu.sync_copy(nodes_v, leaf_hbm.at[pl.ds(base, BW)])
    pltpu.sync_copy(scoresT_v, scores_hbm.at[wid])


def kernel(context_vector, embeddings, thetas):
    ctx_flat = context_vector.reshape(-1).astype(jnp.int32)
    leaf, scores3 = _hs_kernel(ctx_flat, embeddings, thetas)
    scores = scores3.transpose(0, 2, 1).reshape(B, DEPTH)
    return leaf, scores


# speedup vs baseline: 1.7830x; 1.4418x over previous
import functools

import jax
import jax.numpy as jnp
from jax import lax
from jax.experimental import pallas as pl
from jax.experimental.pallas import tpu as pltpu
from jax.experimental.pallas import tpu_sc as plsc

VOCAB = 100000
D = 128
DEPTH = 17
N_INTERNAL = 2**DEPTH - 1
B = 4096
CTX = 8

_INFO = plsc.get_sparse_core_info()
NC, NS, L = _INFO.num_cores, _INFO.num_subcores, _INFO.num_lanes
NW = NC * NS
BW = B // NW
NCHUNK = BW // L
NG = BW // L
DV = D // L
TSTRIDE = L + 1

_mesh = plsc.VectorSubcoreMesh(core_axis_name="c", subcore_axis_name="s")


@functools.partial(
    pl.kernel,
    out_type=[
        jax.ShapeDtypeStruct((B,), jnp.int32),
        jax.ShapeDtypeStruct((NW, DEPTH, BW), jnp.float32),
    ],
    mesh=_mesh,
    compiler_params=pltpu.CompilerParams(needs_layout_passes=False),
    scratch_types=[
        pltpu.VMEM((L * CTX, D), jnp.float32),
        pltpu.VMEM((BW, D), jnp.float32),
        pltpu.VMEM((BW, D), jnp.float32),
        pltpu.VMEM((L * CTX,), jnp.int32),
        pltpu.VMEM((BW,), jnp.int32),
        pltpu.VMEM((DEPTH, BW), jnp.float32),
        pltpu.VMEM((L * TSTRIDE,), jnp.float32),
        pltpu.SemaphoreType.DMA,
    ],
)
def _hs_kernel(ctx_hbm, emb_hbm, th_hbm, leaf_hbm, scores_hbm,
               rows_v, xw_v, theta_v, idx_v, nodes_v, scoresT_v, tbuf_v, sem):
    wid = lax.axis_index("s") * NC + lax.axis_index("c")
    base = pl.multiple_of(wid * BW, BW)
    lanes = jnp.arange(L, dtype=jnp.int32)

    def chunk_body(c, carry):
        off = pl.multiple_of((base + c * L) * CTX, L * CTX)
        pltpu.sync_copy(ctx_hbm.at[pl.ds(off, L * CTX)], idx_v)
        pltpu.async_copy(emb_hbm.at[idx_v], rows_v, sem).wait()
        for e in range(L):
            for k in range(DV):
                ds = pl.ds(k * L, L)
                acc = rows_v[e * CTX, ds]
                for j in range(1, CTX):
                    acc = acc + rows_v[e * CTX + j, ds]
                xw_v[c * L + e, ds] = acc
        return carry

    lax.fori_loop(0, NCHUNK, chunk_body, 0)

    for g in range(NG):
        nodes_v[pl.ds(g * L, L)] = jnp.zeros((L,), jnp.int32)

    def step_body(t, carry):
        pltpu.async_copy(th_hbm.at[nodes_v], theta_v, sem).wait()

        def group_body(g, carry2):
            gds = pl.ds(g * L, L)
            e0 = g * L
            for e in range(L):
                p = []
                for k in range(DV):
                    ds = pl.ds(k * L, L)
                    p.append(xw_v[e0 + e, ds] * theta_v[e0 + e, ds])
                acc = ((p[0] + p[1]) + (p[2] + p[3])) + (
                    (p[4] + p[5]) + (p[6] + p[7]))
                plsc.store_scatter(tbuf_v, [lanes * TSTRIDE + e], acc)
            score = tbuf_v[pl.ds(0, L)]
            for r in range(1, L):
                score = score + tbuf_v[pl.ds(r * TSTRIDE, L)]
            scoresT_v[t, gds] = score
            nd = nodes_v[gds]
            nodes_v[gds] = 2 * nd + jnp.where(score < 0.0, 1, 2)
            return carry2

        return lax.fori_loop(0, NG, group_body, carry)

    lax.fori_loop(0, DEPTH, step_body, 0)

    for g in range(NG):
        gds = pl.ds(g * L, L)
        nodes_v[gds] = nodes_v[gds] - N_INTERNAL
    pltpu.sync_copy(nodes_v, leaf_hbm.at[pl.ds(base, BW)])
    pltpu.sync_copy(scoresT_v, scores_hbm.at[wid])


def kernel(context_vector, embeddings, thetas):
    ctx_flat = context_vector.reshape(-1).astype(jnp.int32)
    leaf, scores3 = _hs_kernel(ctx_flat, embeddings, thetas)
    scores = scores3.transpose(0, 2, 1).reshape(B, DEPTH)
    return leaf, scores
